# baseline (speedup 1.0000x reference)
import functools

import jax
import jax.numpy as jnp
from jax import lax
from jax.experimental import pallas as pl
from jax.experimental.pallas import tpu as pltpu
from jax.experimental.pallas import tpu_sc as plsc

V = 10000
E = 160000
D_NODE = 256
D_EDGE = 16
D = 300
N_LAYERS = 3

DP = 320
DH = DP // 2
NC, NS = 2, 16
VP = 10240
EP = 163840
EDGES_PER_TILE = EP // NS
LANE = 16


def _pad2(a, rows, cols):
    return jnp.pad(a, ((0, rows - a.shape[0]), (0, cols - a.shape[1])))


def _relu_mm_body(a_ref, b_ref, o_ref):
    o_ref[...] = jnp.maximum(
        jnp.dot(a_ref[...], b_ref[...], preferred_element_type=jnp.float32), 0.0)


def _tc_relu_mm(a, b):
    bm = 1024
    k = a.shape[1]
    return pl.pallas_call(
        _relu_mm_body,
        grid=(VP // bm,),
        in_specs=[pl.BlockSpec((bm, k), lambda m: (m, 0)),
                  pl.BlockSpec((k, DP), lambda m: (0, 0))],
        out_specs=pl.BlockSpec((bm, DP), lambda m: (m, 0)),
        out_shape=jax.ShapeDtypeStruct((VP, DP), jnp.float32),
    )(a, b)


def _edge_proj_body(masked, ef_ref, w_ref, b_ref, o_ref):
    a = ef_ref[...]
    o = jnp.dot(a, w_ref[0], preferred_element_type=jnp.float32) + b_ref[0]
    if masked:
        m = (lax.broadcasted_iota(jnp.int32, o.shape, 0)
             + pl.program_id(1) * o.shape[0])
        o = jnp.where(m < E, o, -1e30)
    o_ref[...] = o


def _tc_edge_proj(ef, w, b, masked):
    bm = 2048
    mblocks = EP // bm
    return pl.pallas_call(
        functools.partial(_edge_proj_body, masked),
        grid=(NC, mblocks),
        in_specs=[pl.BlockSpec((bm, D_EDGE), lambda c, m: (m, 0)),
                  pl.BlockSpec((1, D_EDGE, DH), lambda c, m: (c, 0, 0)),
                  pl.BlockSpec((1, 1, DH), lambda c, m: (c, 0, 0))],
        out_specs=pl.BlockSpec((bm, DH), lambda c, m: (c * mblocks + m, 0)),
        out_shape=jax.ShapeDtypeStruct((NC * EP, DH), jnp.float32),
    )(ef, w, b)


def _mm_body(a_ref, b_ref, o_ref):
    o_ref[...] = jnp.dot(a_ref[...], b_ref[0], preferred_element_type=jnp.float32)


def _tc_table_mm(h, w):
    bm = 1024
    mblocks = VP // bm
    return pl.pallas_call(
        _mm_body,
        grid=(NC, mblocks),
        in_specs=[pl.BlockSpec((bm, DP), lambda c, m: (m, 0)),
                  pl.BlockSpec((1, DP, DH), lambda c, m: (c, 0, 0))],
        out_specs=pl.BlockSpec((bm, DH), lambda c, m: (c * mblocks + m, 0)),
        out_shape=jax.ShapeDtypeStruct((NC * VP, DH), jnp.float32),
    )(h, w)


def _update_body(h_ref, hv0_ref, hv1_ref, wh_ref, wv0_ref, wv1_ref, b_ref, o_ref):
    acc = jnp.dot(h_ref[...], wh_ref[...], preferred_element_type=jnp.float32)
    acc = acc + jnp.dot(hv0_ref[...], wv0_ref[...], preferred_element_type=jnp.float32)
    acc = acc + jnp.dot(hv1_ref[...], wv1_ref[...], preferred_element_type=jnp.float32)
    o_ref[...] = jnp.maximum(acc + b_ref[...], 0.0)


def _tc_update(h, hv, wh, wv0, wv1, bn):
    bm = 1024
    mblocks = VP // bm
    return pl.pallas_call(
        _update_body,
        grid=(mblocks,),
        in_specs=[pl.BlockSpec((bm, DP), lambda m: (m, 0)),
                  pl.BlockSpec((bm, DH), lambda m: (m, 0)),
                  pl.BlockSpec((bm, DH), lambda m: (mblocks + m, 0)),
                  pl.BlockSpec((DP, DP), lambda m: (0, 0)),
                  pl.BlockSpec((DH, DP), lambda m: (0, 0)),
                  pl.BlockSpec((DH, DP), lambda m: (0, 0)),
                  pl.BlockSpec((1, DP), lambda m: (0, 0))],
        out_specs=pl.BlockSpec((bm, DP), lambda m: (m, 0)),
        out_shape=jax.ShapeDtypeStruct((VP, DP), jnp.float32),
    )(h, hv, hv, wh, wv0, wv1, bn)


def _final_body(h_ref, w_ref, hn0_ref, hn1_ref, o_ref):
    s = jnp.dot(h_ref[...], w_ref[...], preferred_element_type=jnp.float32)
    o_ref[...] = s * jnp.concatenate([hn0_ref[...], hn1_ref[...]], axis=1)


def _tc_final(h, w, hnbr):
    bm = 1024
    mblocks = VP // bm
    return pl.pallas_call(
        _final_body,
        grid=(mblocks,),
        in_specs=[pl.BlockSpec((bm, DP), lambda m: (m, 0)),
                  pl.BlockSpec((DP, DP), lambda m: (0, 0)),
                  pl.BlockSpec((bm, DH), lambda m: (m, 0)),
                  pl.BlockSpec((bm, DH), lambda m: (mblocks + m, 0))],
        out_specs=pl.BlockSpec((bm, DP), lambda m: (m, 0)),
        out_shape=jax.ShapeDtypeStruct((VP, DP), jnp.float32),
    )(h, w, hnbr, hnbr)


def _make_sc_edge(mode):
    chunk, nb = (80, 2) if mode == "relu_add" else (40, 2)
    nchunk = EDGES_PER_TILE // chunk
    mesh = plsc.VectorSubcoreMesh(
        core_axis_name="c", subcore_axis_name="s", num_cores=NC, num_subcores=NS)

    scratch = [
        pltpu.VMEM_SHARED((V, DH), jnp.float32),
        pltpu.VMEM((nb, 2, chunk), jnp.int32),
        pltpu.VMEM((nb, chunk, DH), jnp.float32),
        pltpu.SemaphoreType.DMA((nb,)),
        pltpu.SemaphoreType.DMA((nb,)),
        pltpu.SemaphoreType.DMA((nb,)),
    ]
    if mode == "mul":
        scratch.insert(3, pltpu.VMEM((nb, chunk, DH), jnp.float32))

    def sc_edge(table, idx2, edata, out, acc, ib, ebuf, *rest):
        if mode == "mul":
            gbuf, sem_e, sem_g, sem_s = rest
        else:
            sem_e, sem_g, sem_s = rest
            gbuf = ebuf
        cid = lax.axis_index("c")
        sid = lax.axis_index("s")
        rows_per_tile = V // NS
        astripe = sid * rows_per_tile

        def iload(g, b):
            return pltpu.make_async_copy(
                idx2.at[cid, sid * nchunk + g], ib.at[b], sem_e.at[b])

        def eload(g, b):
            off = cid * EP + sid * EDGES_PER_TILE + g * chunk
            return pltpu.make_async_copy(
                edata.at[pl.ds(off, chunk), :], ebuf.at[b], sem_e.at[b])

        def gath_start(b):
            pltpu.async_copy(table.at[ib.at[b, 0]], gbuf.at[b], sem_g.at[b],
                             add=(mode == "relu_add"))

        def gath_wait(b):
            pltpu.make_async_copy(
                table.at[ib.at[b, 0]], gbuf.at[b], sem_g.at[b]).wait()

        def scat_start(b):
            pltpu.async_copy(gbuf.at[b], acc.at[ib.at[b, 1]], sem_s.at[b],
                             add=True)

        def scat_wait(b):
            pltpu.make_async_copy(
                gbuf.at[b], acc.at[ib.at[b, 1]], sem_s.at[b]).wait()

        def refill(g, b):
            iload(g, b).start()
            eload(g, b).start()

        def wait_loads(g, b):
            iload(g, b).wait()
            eload(g, b).wait()

        def compute(b):
            def row_body(r, _):
                for cc in range(DH // LANE):
                    sl = pl.ds(cc * LANE, LANE)
                    if mode == "relu_add":
                        ebuf[b, r, sl] = jnp.maximum(ebuf[b, r, sl], 0.0)
                    else:
                        gbuf[b, r, sl] = gbuf[b, r, sl] * ebuf[b, r, sl]
                return 0
            lax.fori_loop(0, chunk, row_body, 0)

        zero = jnp.zeros((LANE,), jnp.float32)

        def zrow(r, _):
            for cc in range(DH // LANE):
                ebuf[0, r, pl.ds(cc * LANE, LANE)] = zero
            return 0
        lax.fori_loop(0, chunk, zrow, 0)
        nfull = rows_per_tile // chunk
        for b in range(nfull):
            pltpu.sync_copy(
                ebuf.at[0], acc.at[pl.ds(astripe + b * chunk, chunk), :])
        rem = rows_per_tile - nfull * chunk
        if rem:
            pltpu.sync_copy(
                ebuf.at[0, pl.ds(0, rem), :],
                acc.at[pl.ds(astripe + nfull * chunk, rem), :])
        plsc.subcore_barrier()

        refill(0, 0)
        refill(1, 1)
        wait_loads(0, 0)
        gath_start(0)

        def outer(go, _):
            for b in range(nb):
                g = go * nb + b
                bn = (b + 1) % nb
                br = (b + 2) % nb

                @pl.when(g + 1 < nchunk)
                def _():
                    wait_loads(g + 1, bn)
                    gath_start(bn)

                gath_wait(b)
                compute(b)
                scat_start(b)

                @pl.when(g + 2 < nchunk)
                def _():
                    @pl.when(g + 2 >= nb)
                    def _():
                        scat_wait(br)
                    refill(g + 2, br)
            return 0
        lax.fori_loop(0, nchunk // nb, outer, 0)

        for b in range(nb):
            scat_wait(b)
        plsc.subcore_barrier()

        pltpu.sync_copy(
            acc.at[pl.ds(astripe, rows_per_tile), :],
            out.at[pl.ds(cid * VP + astripe, rows_per_tile), :])

    return pl.kernel(
        sc_edge,
        out_type=jax.ShapeDtypeStruct((NC * VP, DH), jnp.float32),
        mesh=mesh,
        scratch_types=scratch,
        compiler_params=pltpu.CompilerParams(use_tc_tiling_on_sc=False),
    )


_sc_edge_relu_add = _make_sc_edge("relu_add")
_sc_edge_mul = _make_sc_edge("mul")


def kernel(node_feats, edge_feats, edge_index, W_in, W_msg, b_msg,
           W_new, b_new, W_node, W_edge, W_self):
    f32 = jnp.float32
    src = edge_index[0].astype(jnp.int32)
    dst = edge_index[1].astype(jnp.int32)

    nf = _pad2(node_feats.astype(f32), VP, D_NODE)
    ef = _pad2(edge_feats.astype(f32), EP, D_EDGE)
    srcp = jnp.zeros((EP,), jnp.int32).at[:E].set(src)
    dstp = jnp.zeros((EP,), jnp.int32).at[:E].set(dst)

    def build_idx2(chunk):
        s = srcp.reshape(EP // chunk, chunk)
        d = dstp.reshape(EP // chunk, chunk)
        return jnp.stack(
            [jnp.stack([s + c * VP, d], axis=1) for c in range(NC)])

    idx2_msg = build_idx2(80)
    idx2_cmp = build_idx2(40)

    win_t = _pad2(W_in.T.astype(f32), D_NODE, DP)
    wmh = _pad2(W_msg[:, :D].T.astype(f32), DP, DP)
    wme = _pad2(W_msg[:, D:].T.astype(f32), D_EDGE, DP)
    bmsg = _pad2(b_msg[None, :].astype(f32), 1, DP)
    wnh = _pad2(W_new[:, :D].T.astype(f32), DP, DP)
    wnv = _pad2(W_new[:, D:].T.astype(f32), DP, DP)
    bnew = _pad2(b_new[None, :].astype(f32), 1, DP)
    wnode_t = _pad2(W_node.T.astype(f32), DP, DP)
    wedge_t = _pad2(W_edge.T.astype(f32), D_EDGE, DP)
    wself_t = _pad2(W_self.T.astype(f32), DP, DP)

    def split_cols(w):
        return jnp.stack([w[:, :DH], w[:, DH:]])

    zb = split_cols(jnp.zeros((1, DP), f32))
    eproj = _tc_edge_proj(ef, split_cols(wme), split_cols(bmsg), masked=True)
    hef = _tc_edge_proj(ef, split_cols(wedge_t), zb, masked=False)

    h = _tc_relu_mm(nf, win_t)
    wmh_s = split_cols(wmh)
    for _ in range(N_LAYERS):
        tbl = _tc_table_mm(h, wmh_s)
        hv = _sc_edge_relu_add(tbl, idx2_msg, eproj)
        h = _tc_update(h, hv, wnh, wnv[:DH], wnv[DH:], bnew)

    tbl = _tc_table_mm(h, split_cols(wnode_t))
    hnbr = _sc_edge_mul(tbl, idx2_cmp, hef)
    out = _tc_final(h, wself_t, hnbr)
    return out[:V, :D]


# speedup vs baseline: 1.0534x; 1.0534x over previous
import functools

import jax
import jax.numpy as jnp
from jax import lax
from jax.experimental import pallas as pl
from jax.experimental.pallas import tpu as pltpu
from jax.experimental.pallas import tpu_sc as plsc

V = 10000
E = 160000
D_NODE = 256
D_EDGE = 16
D = 300
N_LAYERS = 3

DP = 320
DH = DP // 2
NC, NS = 2, 16
VP = 10240
EP = 163840
EDGES_PER_TILE = EP // NS
LANE = 16


def _pad2(a, rows, cols):
    return jnp.pad(a, ((0, rows - a.shape[0]), (0, cols - a.shape[1])))


def _relu_mm_body(a_ref, b_ref, o_ref):
    o_ref[...] = jnp.maximum(
        jnp.dot(a_ref[...], b_ref[...], preferred_element_type=jnp.float32), 0.0)


def _tc_relu_mm(a, b):
    bm = 1024
    k = a.shape[1]
    return pl.pallas_call(
        _relu_mm_body,
        grid=(VP // bm,),
        in_specs=[pl.BlockSpec((bm, k), lambda m: (m, 0)),
                  pl.BlockSpec((k, DP), lambda m: (0, 0))],
        out_specs=pl.BlockSpec((bm, DP), lambda m: (m, 0)),
        out_shape=jax.ShapeDtypeStruct((VP, DP), jnp.float32),
    )(a, b)


def _edge_proj_body(ef_ref, wme_ref, bm_ref, wed_ref, ep_ref, hf_ref):
    a = ef_ref[...]
    ep = jnp.dot(a, wme_ref[0], preferred_element_type=jnp.float32) + bm_ref[0]
    m = lax.broadcasted_iota(jnp.int32, ep.shape, 0) + pl.program_id(1) * ep.shape[0]
    ep_ref[...] = jnp.where(m < E, ep, -1e30)
    hf_ref[...] = jnp.dot(a, wed_ref[0], preferred_element_type=jnp.float32)


def _tc_edge_proj(ef, wme, bmsg, wed):
    bm = 2048
    mblocks = EP // bm
    out_sds = jax.ShapeDtypeStruct((NC * EP, DH), jnp.float32)
    return pl.pallas_call(
        _edge_proj_body,
        grid=(NC, mblocks),
        in_specs=[pl.BlockSpec((bm, D_EDGE), lambda c, m: (m, 0)),
                  pl.BlockSpec((1, D_EDGE, DH), lambda c, m: (c, 0, 0)),
                  pl.BlockSpec((1, 1, DH), lambda c, m: (c, 0, 0)),
                  pl.BlockSpec((1, D_EDGE, DH), lambda c, m: (c, 0, 0))],
        out_specs=[pl.BlockSpec((bm, DH), lambda c, m: (c * mblocks + m, 0)),
                   pl.BlockSpec((bm, DH), lambda c, m: (c * mblocks + m, 0))],
        out_shape=[out_sds, out_sds],
    )(ef, wme, bmsg, wed)


def _mm_body(a_ref, b_ref, o_ref):
    o_ref[...] = jnp.dot(a_ref[...], b_ref[0], preferred_element_type=jnp.float32)


def _tc_table_mm(h, w):
    bm = 1024
    mblocks = VP // bm
    return pl.pallas_call(
        _mm_body,
        grid=(NC, mblocks),
        in_specs=[pl.BlockSpec((bm, DP), lambda c, m: (m, 0)),
                  pl.BlockSpec((1, DP, DH), lambda c, m: (c, 0, 0))],
        out_specs=pl.BlockSpec((bm, DH), lambda c, m: (c * mblocks + m, 0)),
        out_shape=jax.ShapeDtypeStruct((NC * VP, DH), jnp.float32),
    )(h, w)


def _update_body(h_ref, hv0_ref, hv1_ref, wh_ref, wv0_ref, wv1_ref, b_ref, o_ref):
    acc = jnp.dot(h_ref[...], wh_ref[...], preferred_element_type=jnp.float32)
    acc = acc + jnp.dot(hv0_ref[...], wv0_ref[...], preferred_element_type=jnp.float32)
    acc = acc + jnp.dot(hv1_ref[...], wv1_ref[...], preferred_element_type=jnp.float32)
    o_ref[...] = jnp.maximum(acc + b_ref[...], 0.0)


def _tc_update(h, hv, wh, wv0, wv1, bn):
    bm = 1024
    mblocks = VP // bm
    return pl.pallas_call(
        _update_body,
        grid=(mblocks,),
        in_specs=[pl.BlockSpec((bm, DP), lambda m: (m, 0)),
                  pl.BlockSpec((bm, DH), lambda m: (m, 0)),
                  pl.BlockSpec((bm, DH), lambda m: (mblocks + m, 0)),
                  pl.BlockSpec((DP, DP), lambda m: (0, 0)),
                  pl.BlockSpec((DH, DP), lambda m: (0, 0)),
                  pl.BlockSpec((DH, DP), lambda m: (0, 0)),
                  pl.BlockSpec((1, DP), lambda m: (0, 0))],
        out_specs=pl.BlockSpec((bm, DP), lambda m: (m, 0)),
        out_shape=jax.ShapeDtypeStruct((VP, DP), jnp.float32),
    )(h, hv, hv, wh, wv0, wv1, bn)


def _final_body(h_ref, w_ref, hn0_ref, hn1_ref, o_ref):
    s = jnp.dot(h_ref[...], w_ref[...], preferred_element_type=jnp.float32)
    o_ref[...] = s * jnp.concatenate([hn0_ref[...], hn1_ref[...]], axis=1)


def _tc_final(h, w, hnbr):
    bm = 1024
    mblocks = VP // bm
    return pl.pallas_call(
        _final_body,
        grid=(mblocks,),
        in_specs=[pl.BlockSpec((bm, DP), lambda m: (m, 0)),
                  pl.BlockSpec((DP, DP), lambda m: (0, 0)),
                  pl.BlockSpec((bm, DH), lambda m: (m, 0)),
                  pl.BlockSpec((bm, DH), lambda m: (mblocks + m, 0))],
        out_specs=pl.BlockSpec((bm, DP), lambda m: (m, 0)),
        out_shape=jax.ShapeDtypeStruct((VP, DP), jnp.float32),
    )(h, w, hnbr, hnbr)


def _make_sc_edge(mode):
    chunk, nb = (80, 2) if mode == "relu_add" else (40, 2)
    nchunk = EDGES_PER_TILE // chunk
    mesh = plsc.VectorSubcoreMesh(
        core_axis_name="c", subcore_axis_name="s", num_cores=NC, num_subcores=NS)

    scratch = [
        pltpu.VMEM_SHARED((V, DH), jnp.float32),
        pltpu.VMEM((nb, 2, chunk), jnp.int32),
        pltpu.VMEM((nb, chunk, DH), jnp.float32),
        pltpu.SemaphoreType.DMA((nb,)),
        pltpu.SemaphoreType.DMA((nb,)),
        pltpu.SemaphoreType.DMA((nb,)),
    ]
    if mode == "mul":
        scratch.insert(3, pltpu.VMEM((nb, chunk, DH), jnp.float32))

    def sc_edge(table, idx2, edata, out, acc, ib, ebuf, *rest):
        if mode == "mul":
            gbuf, sem_e, sem_g, sem_s = rest
        else:
            sem_e, sem_g, sem_s = rest
            gbuf = ebuf
        cid = lax.axis_index("c")
        sid = lax.axis_index("s")
        rows_per_tile = V // NS
        astripe = sid * rows_per_tile

        def iload(g, b):
            return pltpu.make_async_copy(
                idx2.at[cid, sid * nchunk + g], ib.at[b], sem_e.at[b])

        def eload(g, b):
            off = cid * EP + sid * EDGES_PER_TILE + g * chunk
            return pltpu.make_async_copy(
                edata.at[pl.ds(off, chunk), :], ebuf.at[b], sem_e.at[b])

        def gath_start(b):
            pltpu.async_copy(table.at[ib.at[b, 0]], gbuf.at[b], sem_g.at[b],
                             add=(mode == "relu_add"))

        def gath_wait(b):
            pltpu.make_async_copy(
                table.at[ib.at[b, 0]], gbuf.at[b], sem_g.at[b]).wait()

        def scat_start(b):
            pltpu.async_copy(gbuf.at[b], acc.at[ib.at[b, 1]], sem_s.at[b],
                             add=True)

        def scat_wait(b):
            pltpu.make_async_copy(
                gbuf.at[b], acc.at[ib.at[b, 1]], sem_s.at[b]).wait()

        def refill(g, b):
            iload(g, b).start()
            eload(g, b).start()

        def wait_loads(g, b):
            iload(g, b).wait()
            eload(g, b).wait()

        def compute(b):
            def row_body(r, _):
                for cc in range(DH // LANE):
                    sl = pl.ds(cc * LANE, LANE)
                    if mode == "relu_add":
                        ebuf[b, r, sl] = jnp.maximum(ebuf[b, r, sl], 0.0)
                    else:
                        gbuf[b, r, sl] = gbuf[b, r, sl] * ebuf[b, r, sl]
                return 0
            lax.fori_loop(0, chunk, row_body, 0)

        zero = jnp.zeros((LANE,), jnp.float32)

        def zrow(r, _):
            for cc in range(DH // LANE):
                ebuf[0, r, pl.ds(cc * LANE, LANE)] = zero
            return 0
        lax.fori_loop(0, chunk, zrow, 0)
        nfull = rows_per_tile // chunk
        for b in range(nfull):
            pltpu.sync_copy(
                ebuf.at[0], acc.at[pl.ds(astripe + b * chunk, chunk), :])
        rem = rows_per_tile - nfull * chunk
        if rem:
            pltpu.sync_copy(
                ebuf.at[0, pl.ds(0, rem), :],
                acc.at[pl.ds(astripe + nfull * chunk, rem), :])
        plsc.subcore_barrier()

        refill(0, 0)
        refill(1, 1)
        wait_loads(0, 0)
        gath_start(0)

        def outer(go, _):
            for b in range(nb):
                g = go * nb + b
                bn = (b + 1) % nb
                br = (b + 2) % nb

                @pl.when(g + 1 < nchunk)
                def _():
                    wait_loads(g + 1, bn)
                    gath_start(bn)

                gath_wait(b)
                compute(b)
                scat_start(b)

                @pl.when(g + 2 < nchunk)
                def _():
                    @pl.when(g + 2 >= nb)
                    def _():
                        scat_wait(br)
                    refill(g + 2, br)
            return 0
        lax.fori_loop(0, nchunk // nb, outer, 0)

        for b in range(nb):
            scat_wait(b)
        plsc.subcore_barrier()

        pltpu.sync_copy(
            acc.at[pl.ds(astripe, rows_per_tile), :],
            out.at[pl.ds(cid * VP + astripe, rows_per_tile), :])

    return pl.kernel(
        sc_edge,
        out_type=jax.ShapeDtypeStruct((NC * VP, DH), jnp.float32),
        mesh=mesh,
        scratch_types=scratch,
        compiler_params=pltpu.CompilerParams(use_tc_tiling_on_sc=False),
    )


_sc_edge_relu_add = _make_sc_edge("relu_add")
_sc_edge_mul = _make_sc_edge("mul")


def kernel(node_feats, edge_feats, edge_index, W_in, W_msg, b_msg,
           W_new, b_new, W_node, W_edge, W_self):
    f32 = jnp.float32
    src = edge_index[0].astype(jnp.int32)
    dst = edge_index[1].astype(jnp.int32)

    nf = _pad2(node_feats.astype(f32), VP, D_NODE)
    ef = _pad2(edge_feats.astype(f32), EP, D_EDGE)
    srcp = jnp.zeros((EP,), jnp.int32).at[:E].set(src)
    dstp = jnp.zeros((EP,), jnp.int32).at[:E].set(dst)

    def build_idx2(chunk):
        s = srcp.reshape(EP // chunk, chunk)
        d = dstp.reshape(EP // chunk, chunk)
        return jnp.stack(
            [jnp.stack([s + c * VP, d], axis=1) for c in range(NC)])

    idx2_msg = build_idx2(80)
    idx2_cmp = build_idx2(40)

    win_t = _pad2(W_in.T.astype(f32), D_NODE, DP)
    wmh = _pad2(W_msg[:, :D].T.astype(f32), DP, DP)
    wme = _pad2(W_msg[:, D:].T.astype(f32), D_EDGE, DP)
    bmsg = _pad2(b_msg[None, :].astype(f32), 1, DP)
    wnh = _pad2(W_new[:, :D].T.astype(f32), DP, DP)
    wnv = _pad2(W_new[:, D:].T.astype(f32), DP, DP)
    bnew = _pad2(b_new[None, :].astype(f32), 1, DP)
    wnode_t = _pad2(W_node.T.astype(f32), DP, DP)
    wedge_t = _pad2(W_edge.T.astype(f32), D_EDGE, DP)
    wself_t = _pad2(W_self.T.astype(f32), DP, DP)

    def split_cols(w):
        return jnp.stack([w[:, :DH], w[:, DH:]])

    eproj, hef = _tc_edge_proj(ef, split_cols(wme), split_cols(bmsg),
                               split_cols(wedge_t))

    h = _tc_relu_mm(nf, win_t)
    wmh_s = split_cols(wmh)
    for _ in range(N_LAYERS):
        tbl = _tc_table_mm(h, wmh_s)
        hv = _sc_edge_relu_add(tbl, idx2_msg, eproj)
        h = _tc_update(h, hv, wnh, wnv[:DH], wnv[DH:], bnew)

    tbl = _tc_table_mm(h, split_cols(wnode_t))
    hnbr = _sc_edge_mul(tbl, idx2_cmp, hef)
    out = _tc_final(h, wself_t, hnbr)
    return out[:V, :D]
